# Initial kernel scaffold; baseline (speedup 1.0000x reference)
import jax
import jax.numpy as jnp
from jax.experimental import pallas as pl


def kernel(x, edge_index, W0, b0, W1, b1, Wx, bx):
    raise NotImplementedError("write your pallas kernel here")


# speedup vs baseline: 7.0120x; 7.0120x over previous
import functools

import jax
import jax.numpy as jnp
from jax import lax
from jax.experimental import pallas as pl
from jax.experimental.pallas import tpu as pltpu
from jax.experimental.pallas import tpu_sc as plsc

N = 10000
D = 128
E = 320000
NC = 2
NS = 16
NW = NC * NS
BLK = 128
EPT = 10240
KB = EPT // BLK
EPAD = NW * EPT
ACC_ROWS = 10240
ZROWS = ACC_ROWS // NS
OROWS = ACC_ROWS // NS
RB = 2000

_MESH = plsc.VectorSubcoreMesh(core_axis_name="c", subcore_axis_name="s")


@functools.partial(
    pl.kernel,
    out_type=jax.ShapeDtypeStruct((NC, ACC_ROWS, D), jnp.float32),
    mesh=_MESH,
    scratch_types=[
        pltpu.VMEM((KB, BLK), jnp.int32),
        pltpu.VMEM((BLK, D), jnp.float32),
        pltpu.VMEM_SHARED((ACC_ROWS, D), jnp.float32),
    ],
)
def _sc_degree(dst_hbm, ones_hbm, zeros_hbm, out_hbm, dst_v, ones_v, acc):
    c = lax.axis_index("c")
    s = lax.axis_index("s")
    wid = s * NC + c
    pltpu.sync_copy(zeros_hbm, acc.at[pl.ds(s * ZROWS, ZROWS)])
    pltpu.sync_copy(ones_hbm, ones_v)
    pltpu.sync_copy(dst_hbm.at[wid], dst_v)
    plsc.subcore_barrier()

    def body(j, carry):
        pltpu.sync_copy(ones_v, acc.at[dst_v.at[j]], add=True)
        return carry

    lax.fori_loop(0, KB, body, 0)
    plsc.subcore_barrier()
    pltpu.sync_copy(acc.at[pl.ds(s * OROWS, OROWS)],
                    out_hbm.at[c, pl.ds(s * OROWS, OROWS)])


@functools.partial(
    pl.kernel,
    out_type=jax.ShapeDtypeStruct((NC, ACC_ROWS, D), jnp.float32),
    mesh=_MESH,
    scratch_types=[
        pltpu.VMEM((KB, BLK), jnp.int32),
        pltpu.VMEM((KB, BLK), jnp.int32),
        pltpu.VMEM((BLK, D), jnp.float32),
        pltpu.VMEM_SHARED((ACC_ROWS, D), jnp.float32),
    ],
)
def _sc_edge_agg(g_hbm, src_hbm, dst_hbm, zeros_hbm, out_hbm,
                 src_v, dst_v, buf, acc):
    c = lax.axis_index("c")
    s = lax.axis_index("s")
    wid = s * NC + c
    pltpu.sync_copy(zeros_hbm, acc.at[pl.ds(s * ZROWS, ZROWS)])
    pltpu.sync_copy(src_hbm.at[wid], src_v)
    pltpu.sync_copy(dst_hbm.at[wid], dst_v)
    plsc.subcore_barrier()

    def body(j, carry):
        pltpu.sync_copy(g_hbm.at[src_v.at[j]], buf)
        pltpu.sync_copy(buf, acc.at[dst_v.at[j]], add=True)
        return carry

    lax.fori_loop(0, KB, body, 0)
    plsc.subcore_barrier()
    pltpu.sync_copy(acc.at[pl.ds(s * OROWS, OROWS)],
                    out_hbm.at[c, pl.ds(s * OROWS, OROWS)])


def _tc_dis_body(p0_ref, p1_ref, o_ref):
    deg = p0_ref[...][:, 0:1] + p1_ref[...][:, 0:1] + 1.0
    o_ref[...] = lax.rsqrt(deg)


_dis_call = pl.pallas_call(
    _tc_dis_body,
    grid=(N // RB,),
    in_specs=[pl.BlockSpec((RB, D), lambda i: (i, 0)),
              pl.BlockSpec((RB, D), lambda i: (i, 0))],
    out_specs=pl.BlockSpec((RB, 1), lambda i: (i, 0)),
    out_shape=jax.ShapeDtypeStruct((N, 1), jnp.float32),
)


def _tc_mm_body(h_ref, w_ref, dis_ref, o_ref):
    o_ref[...] = jnp.dot(h_ref[...], w_ref[...],
                         preferred_element_type=jnp.float32) * dis_ref[...]


_mm_call = pl.pallas_call(
    _tc_mm_body,
    grid=(N // RB,),
    in_specs=[
        pl.BlockSpec((RB, D), lambda i: (i, 0)),
        pl.BlockSpec((D, D), lambda i: (0, 0)),
        pl.BlockSpec((RB, 1), lambda i: (i, 0)),
    ],
    out_specs=pl.BlockSpec((RB, D), lambda i: (i, 0)),
    out_shape=jax.ShapeDtypeStruct((N, D), jnp.float32),
)


def _relu(z):
    return jnp.maximum(z, 0.0)


def _l2_x15(z):
    nrm = jnp.sqrt(jnp.sum(z * z, axis=1, keepdims=True))
    return z * (1.5 / jnp.maximum(nrm, 1e-12))


def _make_post_mm(act):
    def body(p0_ref, p1_ref, g_ref, dis_ref, b_ref, w_ref, o_ref):
        z = (p0_ref[...] + p1_ref[...] + g_ref[...]) * dis_ref[...] + b_ref[...]
        h = act(z)
        o_ref[...] = jnp.dot(h, w_ref[...],
                             preferred_element_type=jnp.float32) * dis_ref[...]

    return pl.pallas_call(
        body,
        grid=(N // RB,),
        in_specs=[
            pl.BlockSpec((RB, D), lambda i: (i, 0)),
            pl.BlockSpec((RB, D), lambda i: (i, 0)),
            pl.BlockSpec((RB, D), lambda i: (i, 0)),
            pl.BlockSpec((RB, 1), lambda i: (i, 0)),
            pl.BlockSpec((1, D), lambda i: (0, 0)),
            pl.BlockSpec((D, D), lambda i: (0, 0)),
        ],
        out_specs=pl.BlockSpec((RB, D), lambda i: (i, 0)),
        out_shape=jax.ShapeDtypeStruct((N, D), jnp.float32),
    )


_post_relu_mm = _make_post_mm(_relu)
_post_l2_mm = _make_post_mm(_l2_x15)


def _final_body(p0_ref, p1_ref, g_ref, dis_ref, b_ref, o_ref):
    o_ref[...] = (p0_ref[...] + p1_ref[...] + g_ref[...]) * dis_ref[...] + b_ref[...]


_final_call = pl.pallas_call(
    _final_body,
    grid=(N // RB,),
    in_specs=[
        pl.BlockSpec((RB, D), lambda i: (i, 0)),
        pl.BlockSpec((RB, D), lambda i: (i, 0)),
        pl.BlockSpec((RB, D), lambda i: (i, 0)),
        pl.BlockSpec((RB, 1), lambda i: (i, 0)),
        pl.BlockSpec((1, D), lambda i: (0, 0)),
    ],
    out_specs=pl.BlockSpec((RB, D), lambda i: (i, 0)),
    out_shape=jax.ShapeDtypeStruct((N, D), jnp.float32),
)


def kernel(x, edge_index, W0, b0, W1, b1, Wx, bx):
    e = edge_index.astype(jnp.int32)
    pad = EPAD - E
    src = jnp.concatenate([e[0], jnp.zeros((pad,), jnp.int32)]).reshape(NW, KB, BLK)
    dst = jnp.concatenate([e[1], jnp.full((pad,), N, jnp.int32)]).reshape(NW, KB, BLK)
    zeros_d = jnp.zeros((ZROWS, D), jnp.float32)
    ones_d = jnp.ones((BLK, D), jnp.float32)

    degp = _sc_degree(dst, ones_d, zeros_d)
    dis = _dis_call(degp[0], degp[1])

    g0 = _mm_call(x, W0, dis)
    a0 = _sc_edge_agg(g0, src, dst, zeros_d)
    g1 = _post_relu_mm(a0[0], a0[1], g0, dis, b0.reshape(1, D), W1)
    a1 = _sc_edge_agg(g1, src, dst, zeros_d)
    g2 = _post_l2_mm(a1[0], a1[1], g1, dis, b1.reshape(1, D), Wx)
    a2 = _sc_edge_agg(g2, src, dst, zeros_d)
    return _final_call(a2[0], a2[1], g2, dis, bx.reshape(1, D))


# speedup vs baseline: 7.9567x; 1.1347x over previous
import functools

import jax
import jax.numpy as jnp
from jax import lax
from jax.experimental import pallas as pl
from jax.experimental.pallas import tpu as pltpu
from jax.experimental.pallas import tpu_sc as plsc

N = 10000
D = 128
E = 320000
NC = 2
NS = 16
NW = NC * NS
BLK = 128
EPT = 10240
KB = EPT // BLK
EPAD = NW * EPT
ACC_ROWS = 10240
ZROWS = ACC_ROWS // NS
OROWS = ACC_ROWS // NS
RB = 2000

_MESH = plsc.VectorSubcoreMesh(core_axis_name="c", subcore_axis_name="s")


@functools.partial(
    pl.kernel,
    out_type=jax.ShapeDtypeStruct((NC, ACC_ROWS, D), jnp.float32),
    mesh=_MESH,
    scratch_types=[
        pltpu.VMEM((KB, BLK), jnp.int32),
        pltpu.VMEM((BLK, D), jnp.float32),
        pltpu.VMEM_SHARED((ACC_ROWS, D), jnp.float32),
    ],
)
def _sc_degree(dst_hbm, ones_hbm, zeros_hbm, out_hbm, dst_v, ones_v, acc):
    c = lax.axis_index("c")
    s = lax.axis_index("s")
    wid = s * NC + c
    pltpu.sync_copy(zeros_hbm, acc.at[pl.ds(s * ZROWS, ZROWS)])
    pltpu.sync_copy(ones_hbm, ones_v)
    pltpu.sync_copy(dst_hbm.at[wid], dst_v)
    plsc.subcore_barrier()

    def body(j, carry):
        pltpu.sync_copy(ones_v, acc.at[dst_v.at[j]], add=True)
        return carry

    lax.fori_loop(0, KB, body, 0)
    plsc.subcore_barrier()
    pltpu.sync_copy(acc.at[pl.ds(s * OROWS, OROWS)],
                    out_hbm.at[c, pl.ds(s * OROWS, OROWS)])


NBUF = 2
NHALF = 2
HKB = KB // NHALF


@functools.partial(
    pl.kernel,
    out_type=jax.ShapeDtypeStruct((NC, ACC_ROWS, D), jnp.float32),
    mesh=_MESH,
    scratch_types=[
        pltpu.VMEM((HKB, BLK), jnp.int32),
        pltpu.VMEM((HKB, BLK), jnp.int32),
        pltpu.VMEM((BLK, D), jnp.float32),
        pltpu.VMEM((BLK, D), jnp.float32),
        pltpu.SemaphoreType.DMA,
        pltpu.SemaphoreType.DMA,
        pltpu.VMEM_SHARED((ACC_ROWS, D), jnp.float32),
    ],
)
def _sc_edge_agg(g_hbm, src_hbm, dst_hbm, zeros_hbm, out_hbm,
                 src_v, dst_v, b0, b1, s0, s1, acc):
    bufs = (b0, b1)
    sems = (s0, s1)
    c = lax.axis_index("c")
    s = lax.axis_index("s")
    wid = s * NC + c
    pltpu.sync_copy(zeros_hbm, acc.at[pl.ds(s * ZROWS, ZROWS)])
    plsc.subcore_barrier()

    for h in range(NHALF):
        pltpu.sync_copy(src_hbm.at[wid, pl.ds(h * HKB, HKB)], src_v)
        pltpu.sync_copy(dst_hbm.at[wid, pl.ds(h * HKB, HKB)], dst_v)
        for b in range(NBUF):
            pltpu.async_copy(g_hbm.at[src_v.at[b]], bufs[b], sems[b])

        def body(jj, carry):
            for b in range(NBUF):
                j = jj * NBUF + b
                pltpu.make_async_copy(g_hbm.at[src_v.at[j]], bufs[b],
                                      sems[b]).wait()
                pltpu.sync_copy(bufs[b], acc.at[dst_v.at[j]], add=True)
                nxt = j + NBUF

                @pl.when(nxt < HKB)
                def _():
                    pltpu.async_copy(g_hbm.at[src_v.at[nxt]], bufs[b], sems[b])
            return carry

        lax.fori_loop(0, HKB // NBUF, body, 0)

    plsc.subcore_barrier()
    pltpu.sync_copy(acc.at[pl.ds(s * OROWS, OROWS)],
                    out_hbm.at[c, pl.ds(s * OROWS, OROWS)])


def _tc_dis_body(p0_ref, p1_ref, o_ref):
    deg = p0_ref[...][:, 0:1] + p1_ref[...][:, 0:1] + 1.0
    o_ref[...] = lax.rsqrt(deg)


_dis_call = pl.pallas_call(
    _tc_dis_body,
    grid=(N // RB,),
    in_specs=[pl.BlockSpec((RB, D), lambda i: (i, 0)),
              pl.BlockSpec((RB, D), lambda i: (i, 0))],
    out_specs=pl.BlockSpec((RB, 1), lambda i: (i, 0)),
    out_shape=jax.ShapeDtypeStruct((N, 1), jnp.float32),
)


def _tc_mm_body(h_ref, w_ref, dis_ref, o_ref):
    o_ref[...] = jnp.dot(h_ref[...], w_ref[...],
                         preferred_element_type=jnp.float32) * dis_ref[...]


_mm_call = pl.pallas_call(
    _tc_mm_body,
    grid=(N // RB,),
    in_specs=[
        pl.BlockSpec((RB, D), lambda i: (i, 0)),
        pl.BlockSpec((D, D), lambda i: (0, 0)),
        pl.BlockSpec((RB, 1), lambda i: (i, 0)),
    ],
    out_specs=pl.BlockSpec((RB, D), lambda i: (i, 0)),
    out_shape=jax.ShapeDtypeStruct((N, D), jnp.float32),
)


def _relu(z):
    return jnp.maximum(z, 0.0)


def _l2_x15(z):
    nrm = jnp.sqrt(jnp.sum(z * z, axis=1, keepdims=True))
    return z * (1.5 / jnp.maximum(nrm, 1e-12))


def _make_post_mm(act):
    def body(p0_ref, p1_ref, g_ref, dis_ref, b_ref, w_ref, o_ref):
        z = (p0_ref[...] + p1_ref[...] + g_ref[...]) * dis_ref[...] + b_ref[...]
        h = act(z)
        o_ref[...] = jnp.dot(h, w_ref[...],
                             preferred_element_type=jnp.float32) * dis_ref[...]

    return pl.pallas_call(
        body,
        grid=(N // RB,),
        in_specs=[
            pl.BlockSpec((RB, D), lambda i: (i, 0)),
            pl.BlockSpec((RB, D), lambda i: (i, 0)),
            pl.BlockSpec((RB, D), lambda i: (i, 0)),
            pl.BlockSpec((RB, 1), lambda i: (i, 0)),
            pl.BlockSpec((1, D), lambda i: (0, 0)),
            pl.BlockSpec((D, D), lambda i: (0, 0)),
        ],
        out_specs=pl.BlockSpec((RB, D), lambda i: (i, 0)),
        out_shape=jax.ShapeDtypeStruct((N, D), jnp.float32),
    )


_post_relu_mm = _make_post_mm(_relu)
_post_l2_mm = _make_post_mm(_l2_x15)


def _final_body(p0_ref, p1_ref, g_ref, dis_ref, b_ref, o_ref):
    o_ref[...] = (p0_ref[...] + p1_ref[...] + g_ref[...]) * dis_ref[...] + b_ref[...]


_final_call = pl.pallas_call(
    _final_body,
    grid=(N // RB,),
    in_specs=[
        pl.BlockSpec((RB, D), lambda i: (i, 0)),
        pl.BlockSpec((RB, D), lambda i: (i, 0)),
        pl.BlockSpec((RB, D), lambda i: (i, 0)),
        pl.BlockSpec((RB, 1), lambda i: (i, 0)),
        pl.BlockSpec((1, D), lambda i: (0, 0)),
    ],
    out_specs=pl.BlockSpec((RB, D), lambda i: (i, 0)),
    out_shape=jax.ShapeDtypeStruct((N, D), jnp.float32),
)


def kernel(x, edge_index, W0, b0, W1, b1, Wx, bx):
    e = edge_index.astype(jnp.int32)
    pad = EPAD - E
    src = jnp.concatenate([e[0], jnp.zeros((pad,), jnp.int32)]).reshape(NW, KB, BLK)
    dst = jnp.concatenate([e[1], jnp.full((pad,), N, jnp.int32)]).reshape(NW, KB, BLK)
    zeros_d = jnp.zeros((ZROWS, D), jnp.float32)
    ones_d = jnp.ones((BLK, D), jnp.float32)

    degp = _sc_degree(dst, ones_d, zeros_d)
    dis = _dis_call(degp[0], degp[1])

    g0 = _mm_call(x, W0, dis)
    a0 = _sc_edge_agg(g0, src, dst, zeros_d)
    g1 = _post_relu_mm(a0[0], a0[1], g0, dis, b0.reshape(1, D), W1)
    a1 = _sc_edge_agg(g1, src, dst, zeros_d)
    g2 = _post_l2_mm(a1[0], a1[1], g1, dis, b1.reshape(1, D), Wx)
    a2 = _sc_edge_agg(g2, src, dst, zeros_d)
    return _final_call(a2[0], a2[1], g2, dis, bx.reshape(1, D))
